# baseline (speedup 1.0000x reference)
import functools

import numpy as np
import jax
import jax.numpy as jnp
from jax import lax
from jax.experimental import pallas as pl
from jax.experimental.pallas import tpu as pltpu
from jax.experimental.pallas import tpu_sc as plsc

N = 10000
E = 320000
EP = 327680
F_IN = 128
H1 = 64
H2 = 128
C = 10
CP = 16
G = 64

NC = 2
NS = 16
NW = NC * NS
K = 128
EROWS = EP // K
NBUF = 5
DIST = 3
NPAD = 10240
ROWS_PT = NPAD // NS
BLK = 512
BN_S = float(1.0 / np.sqrt(1.0 + 1e-5))

_MESH = plsc.VectorSubcoreMesh(core_axis_name="c", subcore_axis_name="s")
_SC_PARAMS = pltpu.CompilerParams(use_tc_tiling_on_sc=False)


def _make_prop(f_band, feat_split):
    steps = (EP // NC if not feat_split else EP) // (NS * K)

    @functools.partial(
        pl.kernel,
        out_type=jax.ShapeDtypeStruct((NC, NPAD, f_band), jnp.float32),
        mesh=_MESH,
        scratch_types=[
            pltpu.VMEM((steps, K), jnp.int32),
            pltpu.VMEM((steps, K), jnp.int32),
            pltpu.VMEM((NBUF, K, f_band), jnp.float32),
            pltpu.VMEM((64, f_band), jnp.float32),
            pltpu.VMEM_SHARED((NPAD, f_band), jnp.float32),
        ] + [pltpu.SemaphoreType.DMA] * (2 * NBUF),
        compiler_params=_SC_PARAMS,
    )
    def prop(u_hbm, edge_hbm, out_hbm, sidx, didx, ring, zbuf, acc, *sems):
        gsems = sems[:NBUF]
        ssems = sems[NBUF:]
        c = lax.axis_index("c")
        t = lax.axis_index("s")
        if feat_split:
            wrow = t * steps
            table = u_hbm.at[c]
        else:
            wrow = (c * NS + t) * steps
            table = u_hbm

        pltpu.sync_copy(edge_hbm.at[0, pl.ds(wrow, steps)], sidx)
        pltpu.sync_copy(edge_hbm.at[1, pl.ds(wrow, steps)], didx)

        zv = jnp.zeros((16,), jnp.float32)

        def zb(i, carry):
            for j in range(f_band // 16):
                zbuf[i, pl.ds(j * 16, 16)] = zv
            return carry

        lax.fori_loop(0, 64, zb, 0)
        for kk in range(ROWS_PT // 64):
            pltpu.sync_copy(zbuf, acc.at[pl.ds(t * ROWS_PT + kk * 64, 64)])
        plsc.subcore_barrier()

        def gather_start(step, b):
            pltpu.async_copy(table.at[sidx.at[step]], ring.at[b], gsems[b])

        def gather_wait(b):
            pltpu.make_async_copy(table.at[sidx.at[0]], ring.at[b],
                                  gsems[b]).wait()

        def scat_start(step, b):
            pltpu.async_copy(ring.at[b], acc.at[didx.at[step]], ssems[b],
                             add=True)

        def scat_wait(b):
            pltpu.make_async_copy(ring.at[b], acc.at[didx.at[0]],
                                  ssems[b]).wait()

        def outer(g, carry):
            for b in range(NBUF):
                i = g * NBUF + b

                @pl.when(g > 0)
                def _():
                    scat_wait(b)

                gather_start(i, b)
                sb = (b - DIST) % NBUF
                if b >= DIST:
                    gather_wait(sb)
                    scat_start(i - DIST, sb)
                else:
                    @pl.when(g > 0)
                    def _():
                        gather_wait(sb)
                        scat_start(i - DIST, sb)
            return carry

        lax.fori_loop(0, steps // NBUF, outer, 0)
        for j in range(steps - DIST, steps):
            b = j % NBUF
            gather_wait(b)
            scat_start(j, b)
        for b in range(NBUF):
            scat_wait(b)
        plsc.subcore_barrier()
        pltpu.sync_copy(acc.at[pl.ds(t * ROWS_PT, ROWS_PT)],
                        out_hbm.at[c, pl.ds(t * ROWS_PT, ROWS_PT)])

    return prop


_PROP1 = _make_prop(H1 // 2, True)
_PROP2 = _make_prop(H2 // 2, True)
_PROP3 = _make_prop(CP, False)

_DEG_STEPS = EP // (NC * NS * K)


@functools.partial(
    pl.kernel,
    out_type=jax.ShapeDtypeStruct((NC, NPAD, CP), jnp.float32),
    mesh=_MESH,
    scratch_types=[
        pltpu.VMEM((_DEG_STEPS, K), jnp.int32),
        pltpu.VMEM((K, CP), jnp.float32),
        pltpu.VMEM((64, CP), jnp.float32),
        pltpu.VMEM_SHARED((NPAD, CP), jnp.float32),
    ] + [pltpu.SemaphoreType.DMA] * NBUF,
    compiler_params=_SC_PARAMS,
)
def _deg(edge_hbm, out_hbm, didx, ones_b, zbuf, acc, *ssems):
    c = lax.axis_index("c")
    t = lax.axis_index("s")
    wrow = (c * NS + t) * _DEG_STEPS
    pltpu.sync_copy(edge_hbm.at[1, pl.ds(wrow, _DEG_STEPS)], didx)

    ones16 = jnp.ones((16,), jnp.float32)
    zv = jnp.zeros((16,), jnp.float32)

    def fill(i, carry):
        ones_b[i, pl.ds(0, 16)] = ones16
        return carry

    lax.fori_loop(0, K, fill, 0)

    def zb(i, carry):
        zbuf[i, pl.ds(0, 16)] = zv
        return carry

    lax.fori_loop(0, 64, zb, 0)
    for kk in range(ROWS_PT // 64):
        pltpu.sync_copy(zbuf, acc.at[pl.ds(t * ROWS_PT + kk * 64, 64)])
    plsc.subcore_barrier()

    def scat_wait(b):
        pltpu.make_async_copy(ones_b, acc.at[didx.at[0]], ssems[b]).wait()

    def outer(g, carry):
        for b in range(NBUF):
            @pl.when(g > 0)
            def _():
                scat_wait(b)

            pltpu.async_copy(ones_b, acc.at[didx.at[g * NBUF + b]], ssems[b],
                             add=True)
        return carry

    lax.fori_loop(0, _DEG_STEPS // NBUF, outer, 0)
    for b in range(NBUF):
        scat_wait(b)
    plsc.subcore_barrier()
    pltpu.sync_copy(acc.at[pl.ds(t * ROWS_PT, ROWS_PT)],
                    out_hbm.at[c, pl.ds(t * ROWS_PT, ROWS_PT)])


_NBLK = NPAD // BLK


def _tc0(xp, W1, g1r, degp):
    def body(x_ref, w_ref, g_ref, d_ref, u1_ref, dinv_ref):
        deg = d_ref[0, :, 0:1] + d_ref[1, :, 0:1] + 1.0
        dinv = lax.rsqrt(deg)
        dinv_ref[...] = dinv
        ws = w_ref[...] * (BN_S * g_ref[...])
        u1 = dinv * jnp.dot(x_ref[...], ws,
                            preferred_element_type=jnp.float32)
        u1_ref[0] = u1[:, :H1 // 2]
        u1_ref[1] = u1[:, H1 // 2:]

    return pl.pallas_call(
        body,
        grid=(_NBLK,),
        in_specs=[
            pl.BlockSpec((BLK, F_IN), lambda i: (i, 0)),
            pl.BlockSpec((F_IN, H1), lambda i: (0, 0)),
            pl.BlockSpec((1, H1), lambda i: (0, 0)),
            pl.BlockSpec((NC, BLK, CP), lambda i: (0, i, 0)),
        ],
        out_specs=(pl.BlockSpec((NC, BLK, H1 // 2), lambda i: (0, i, 0)),
                   pl.BlockSpec((BLK, 1), lambda i: (i, 0))),
        out_shape=(jax.ShapeDtypeStruct((NC, NPAD, H1 // 2), jnp.float32),
                   jax.ShapeDtypeStruct((NPAD, 1), jnp.float32)),
    )(xp, W1, g1r, degp)


def _tc1(p1, u1, dinv, b1r, g1r, be1r, W2, g2r):
    def body(p_ref, u_ref, d_ref, b_ref, g_ref, be_ref, w_ref, gn_ref,
             out_ref):
        dinv = d_ref[...]
        bia = b_ref[...] * (BN_S * g_ref[...]) + be_ref[...]
        full = jnp.concatenate([p_ref[0] + u_ref[0], p_ref[1] + u_ref[1]],
                               axis=1)
        z = jnp.maximum(dinv * full + bia, 0.0)
        ws = w_ref[...] * (BN_S * gn_ref[...])
        u2 = dinv * jnp.dot(z, ws, preferred_element_type=jnp.float32)
        out_ref[0] = u2[:, :H2 // 2]
        out_ref[1] = u2[:, H2 // 2:]

    return pl.pallas_call(
        body,
        grid=(_NBLK,),
        in_specs=[
            pl.BlockSpec((NC, BLK, H1 // 2), lambda i: (0, i, 0)),
            pl.BlockSpec((NC, BLK, H1 // 2), lambda i: (0, i, 0)),
            pl.BlockSpec((BLK, 1), lambda i: (i, 0)),
            pl.BlockSpec((1, H1), lambda i: (0, 0)),
            pl.BlockSpec((1, H1), lambda i: (0, 0)),
            pl.BlockSpec((1, H1), lambda i: (0, 0)),
            pl.BlockSpec((H1, H2), lambda i: (0, 0)),
            pl.BlockSpec((1, H2), lambda i: (0, 0)),
        ],
        out_specs=pl.BlockSpec((NC, BLK, H2 // 2), lambda i: (0, i, 0)),
        out_shape=jax.ShapeDtypeStruct((NC, NPAD, H2 // 2), jnp.float32),
    )(p1, u1, dinv, b1r, g1r, be1r, W2, g2r)


def _tc2(p2, u2, dinv, b2r, g2r, be2r, W3p):
    def body(p_ref, u_ref, d_ref, b_ref, g_ref, be_ref, w_ref, out_ref):
        dinv = d_ref[...]
        bia = b_ref[...] * (BN_S * g_ref[...]) + be_ref[...]
        full = jnp.concatenate([p_ref[0] + u_ref[0], p_ref[1] + u_ref[1]],
                               axis=1)
        z = jnp.maximum(dinv * full + bia, 0.0)
        out_ref[...] = dinv * jnp.dot(z, w_ref[...],
                                      preferred_element_type=jnp.float32)

    return pl.pallas_call(
        body,
        grid=(_NBLK,),
        in_specs=[
            pl.BlockSpec((NC, BLK, H2 // 2), lambda i: (0, i, 0)),
            pl.BlockSpec((NC, BLK, H2 // 2), lambda i: (0, i, 0)),
            pl.BlockSpec((BLK, 1), lambda i: (i, 0)),
            pl.BlockSpec((1, H2), lambda i: (0, 0)),
            pl.BlockSpec((1, H2), lambda i: (0, 0)),
            pl.BlockSpec((1, H2), lambda i: (0, 0)),
            pl.BlockSpec((H2, CP), lambda i: (0, 0)),
        ],
        out_specs=pl.BlockSpec((BLK, CP), lambda i: (i, 0)),
        out_shape=jax.ShapeDtypeStruct((NPAD, CP), jnp.float32),
    )(p2, u2, dinv, b2r, g2r, be2r, W3p)


_PBLK = 1000
_PNBLK = N // _PBLK


def _tc3(p3, u3, dinv, b3r, batch2d):
    def body(p_ref, u_ref, d_ref, b_ref, bt_ref, out_ref, acc, cacc):
        i = pl.program_id(0)

        @pl.when(i == 0)
        def _():
            acc[...] = jnp.zeros_like(acc)
            cacc[...] = jnp.zeros_like(cacc)

        dinv = d_ref[...]
        h3 = dinv * (p_ref[0] + p_ref[1] + u_ref[...]) + b_ref[...]
        oh = (bt_ref[...] == lax.broadcasted_iota(jnp.int32, (_PBLK, G), 1)
              ).astype(jnp.float32)
        acc[...] += lax.dot_general(oh, h3, (((0,), (0,)), ((), ())),
                                    preferred_element_type=jnp.float32)
        cacc[...] += lax.dot_general(oh, jnp.ones((_PBLK, 8), jnp.float32),
                                     (((0,), (0,)), ((), ())),
                                     preferred_element_type=jnp.float32)

        @pl.when(i == _PNBLK - 1)
        def _():
            pooled = acc[...] / jnp.maximum(cacc[:, 0:1], 1.0)
            valid = lax.broadcasted_iota(jnp.int32, (G, CP), 1) < C
            m = jnp.max(jnp.where(valid, pooled, -1e30), axis=1,
                        keepdims=True)
            ex = jnp.where(valid, jnp.exp(pooled - m), 0.0)
            lse = jnp.log(jnp.sum(ex, axis=1, keepdims=True))
            out_ref[...] = pooled - m - lse

    return pl.pallas_call(
        body,
        grid=(_PNBLK,),
        in_specs=[
            pl.BlockSpec((NC, _PBLK, CP), lambda i: (0, i, 0)),
            pl.BlockSpec((_PBLK, CP), lambda i: (i, 0)),
            pl.BlockSpec((_PBLK, 1), lambda i: (i, 0)),
            pl.BlockSpec((1, CP), lambda i: (0, 0)),
            pl.BlockSpec((_PBLK, 1), lambda i: (i, 0)),
        ],
        out_specs=pl.BlockSpec((G, CP), lambda i: (0, 0)),
        out_shape=jax.ShapeDtypeStruct((G, CP), jnp.float32),
        scratch_shapes=[pltpu.VMEM((G, CP), jnp.float32),
                        pltpu.VMEM((G, 8), jnp.float32)],
    )(p3, u3, dinv, b3r, batch2d)


def kernel(x, edge_index, batch, W1, b1, g1, be1, W2, b2, g2, be2, W3, b3):
    xpad = jnp.pad(x, ((0, NPAD - N), (0, 0)))
    edge3d = jnp.pad(edge_index, ((0, 0), (0, EP - E)),
                     constant_values=N).reshape(2, EROWS, K)
    batch2d = batch.reshape(N, 1)
    W3pad = jnp.pad(W3, ((0, 0), (0, CP - C)))
    b3r = jnp.pad(b3, (0, CP - C)).reshape(1, CP)
    b1r, g1r, be1r = b1.reshape(1, H1), g1.reshape(1, H1), be1.reshape(1, H1)
    b2r, g2r, be2r = b2.reshape(1, H2), g2.reshape(1, H2), be2.reshape(1, H2)

    degp = _deg(edge3d)
    u1, dinv = _tc0(xpad, W1, g1r, degp)
    p1 = _PROP1(u1, edge3d)
    u2 = _tc1(p1, u1, dinv, b1r, g1r, be1r, W2, g2r)
    p2 = _PROP2(u2, edge3d)
    u3 = _tc2(p2, u2, dinv, b2r, g2r, be2r, W3pad)
    p3 = _PROP3(u3, edge3d)
    out16 = _tc3(p3, u3, dinv, b3r, batch2d)
    return out16[:, :C]


# speedup vs baseline: 2.1639x; 2.1639x over previous
import functools

import numpy as np
import jax
import jax.numpy as jnp
from jax import lax
from jax.experimental import pallas as pl
from jax.experimental.pallas import tpu as pltpu
from jax.experimental.pallas import tpu_sc as plsc

N = 10000
E = 320000
EP = 327680
F_IN = 128
H1 = 64
H2 = 128
C = 10
CP = 16
G = 64

NC = 2
NS = 16
NW = NC * NS
K = 128
EROWS = EP // K
NBUF = 5
DIST = 3
NPAD = 10240
ROWS_PT = NPAD // NS
BLK = 512
BN_S = float(1.0 / np.sqrt(1.0 + 1e-5))

_MESH = plsc.VectorSubcoreMesh(core_axis_name="c", subcore_axis_name="s")
_SC_PARAMS = pltpu.CompilerParams(use_tc_tiling_on_sc=False)


def _make_prop(f_band, feat_split):
    steps = (EP // NC if not feat_split else EP) // (NS * K)

    @functools.partial(
        pl.kernel,
        out_type=jax.ShapeDtypeStruct((NC, NPAD, f_band), jnp.float32),
        mesh=_MESH,
        scratch_types=[
            pltpu.VMEM((steps, K), jnp.int32),
            pltpu.VMEM((steps, K), jnp.int32),
            pltpu.VMEM((NBUF, K, f_band), jnp.float32),
            pltpu.VMEM((64, f_band), jnp.float32),
            pltpu.VMEM_SHARED((NPAD, f_band), jnp.float32),
        ] + [pltpu.SemaphoreType.DMA] * (2 * NBUF),
        compiler_params=_SC_PARAMS,
    )
    def prop(u_hbm, edge_hbm, out_hbm, sidx, didx, ring, zbuf, acc, *sems):
        gsems = sems[:NBUF]
        ssems = sems[NBUF:]
        c = lax.axis_index("c")
        t = lax.axis_index("s")
        if feat_split:
            wrow = t * steps
            table = u_hbm.at[c]
        else:
            wrow = (c * NS + t) * steps
            table = u_hbm

        pltpu.sync_copy(edge_hbm.at[0, pl.ds(wrow, steps)], sidx)
        pltpu.sync_copy(edge_hbm.at[1, pl.ds(wrow, steps)], didx)

        zv = jnp.zeros((16,), jnp.float32)

        def zb(i, carry):
            for j in range(f_band // 16):
                zbuf[i, pl.ds(j * 16, 16)] = zv
            return carry

        lax.fori_loop(0, 64, zb, 0)
        for kk in range(ROWS_PT // 64):
            pltpu.sync_copy(zbuf, acc.at[pl.ds(t * ROWS_PT + kk * 64, 64)])
        plsc.subcore_barrier()

        def gather_start(step, b):
            pltpu.async_copy(table.at[sidx.at[step]], ring.at[b], gsems[b])

        def gather_wait(b):
            pltpu.make_async_copy(table.at[sidx.at[0]], ring.at[b],
                                  gsems[b]).wait()

        def scat_start(step, b):
            pltpu.async_copy(ring.at[b], acc.at[didx.at[step]], ssems[b],
                             add=True)

        def scat_wait(b):
            pltpu.make_async_copy(ring.at[b], acc.at[didx.at[0]],
                                  ssems[b]).wait()

        def outer(g, carry):
            for b in range(NBUF):
                i = g * NBUF + b

                @pl.when(g > 0)
                def _():
                    scat_wait(b)

                gather_start(i, b)
                sb = (b - DIST) % NBUF
                if b >= DIST:
                    gather_wait(sb)
                    scat_start(i - DIST, sb)
                else:
                    @pl.when(g > 0)
                    def _():
                        gather_wait(sb)
                        scat_start(i - DIST, sb)
            return carry

        lax.fori_loop(0, steps // NBUF, outer, 0)
        for j in range(steps - DIST, steps):
            b = j % NBUF
            gather_wait(b)
            scat_start(j, b)
        for b in range(NBUF):
            scat_wait(b)
        plsc.subcore_barrier()
        pltpu.sync_copy(acc.at[pl.ds(t * ROWS_PT, ROWS_PT)],
                        out_hbm.at[c, pl.ds(t * ROWS_PT, ROWS_PT)])

    return prop


_PROP1 = _make_prop(H1 // 2, True)
_PROP2 = _make_prop(H2 // 2, True)
_PROP3 = _make_prop(CP, False)

_DEG_STEPS = EP // (NC * NS * K)


@functools.partial(
    pl.kernel,
    out_type=jax.ShapeDtypeStruct((NC, NPAD, CP), jnp.float32),
    mesh=_MESH,
    scratch_types=[
        pltpu.VMEM((_DEG_STEPS, K), jnp.int32),
        pltpu.VMEM((K, CP), jnp.float32),
        pltpu.VMEM((64, CP), jnp.float32),
        pltpu.VMEM_SHARED((NPAD, CP), jnp.float32),
    ] + [pltpu.SemaphoreType.DMA] * NBUF,
    compiler_params=_SC_PARAMS,
)
def _deg(edge_hbm, out_hbm, didx, ones_b, zbuf, acc, *ssems):
    c = lax.axis_index("c")
    t = lax.axis_index("s")
    wrow = (c * NS + t) * _DEG_STEPS
    pltpu.sync_copy(edge_hbm.at[1, pl.ds(wrow, _DEG_STEPS)], didx)

    ones16 = jnp.ones((16,), jnp.float32)
    zv = jnp.zeros((16,), jnp.float32)

    def fill(i, carry):
        ones_b[i, pl.ds(0, 16)] = ones16
        return carry

    lax.fori_loop(0, K, fill, 0)

    def zb(i, carry):
        zbuf[i, pl.ds(0, 16)] = zv
        return carry

    lax.fori_loop(0, 64, zb, 0)
    for kk in range(ROWS_PT // 64):
        pltpu.sync_copy(zbuf, acc.at[pl.ds(t * ROWS_PT + kk * 64, 64)])
    plsc.subcore_barrier()

    def scat_wait(b):
        pltpu.make_async_copy(ones_b, acc.at[didx.at[0]], ssems[b]).wait()

    def outer(g, carry):
        for b in range(NBUF):
            @pl.when(g > 0)
            def _():
                scat_wait(b)

            pltpu.async_copy(ones_b, acc.at[didx.at[g * NBUF + b]], ssems[b],
                             add=True)
        return carry

    lax.fori_loop(0, _DEG_STEPS // NBUF, outer, 0)
    for b in range(NBUF):
        scat_wait(b)
    plsc.subcore_barrier()
    pltpu.sync_copy(acc.at[pl.ds(t * ROWS_PT, ROWS_PT)],
                    out_hbm.at[c, pl.ds(t * ROWS_PT, ROWS_PT)])


_NBLK = NPAD // BLK


def _tc0(xp, W1, g1r, degp):
    def body(x_ref, w_ref, g_ref, d_ref, u1_ref, dinv_ref):
        deg = d_ref[0, :, 0:1] + d_ref[1, :, 0:1] + 1.0
        dinv = lax.rsqrt(deg)
        dinv_ref[...] = dinv
        ws = w_ref[...] * (BN_S * g_ref[...])
        u1 = dinv * jnp.dot(x_ref[...], ws,
                            preferred_element_type=jnp.float32)
        u1_ref[0] = u1[:, :H1 // 2]
        u1_ref[1] = u1[:, H1 // 2:]

    return pl.pallas_call(
        body,
        out_shape=(jax.ShapeDtypeStruct((NC, NPAD, H1 // 2), jnp.float32),
                   jax.ShapeDtypeStruct((NPAD, 1), jnp.float32)),
    )(xp, W1, g1r, degp)


def _tc1(p1, u1, dinv, b1r, g1r, be1r, W2, g2r):
    def body(p_ref, u_ref, d_ref, b_ref, g_ref, be_ref, w_ref, gn_ref,
             out_ref):
        dinv = d_ref[...]
        bia = b_ref[...] * (BN_S * g_ref[...]) + be_ref[...]
        full = jnp.concatenate([p_ref[0] + u_ref[0], p_ref[1] + u_ref[1]],
                               axis=1)
        z = jnp.maximum(dinv * full + bia, 0.0)
        ws = w_ref[...] * (BN_S * gn_ref[...])
        u2 = dinv * jnp.dot(z, ws, preferred_element_type=jnp.float32)
        out_ref[0] = u2[:, :H2 // 2]
        out_ref[1] = u2[:, H2 // 2:]

    return pl.pallas_call(
        body,
        out_shape=jax.ShapeDtypeStruct((NC, NPAD, H2 // 2), jnp.float32),
    )(p1, u1, dinv, b1r, g1r, be1r, W2, g2r)


def _tc2(p2, u2, dinv, b2r, g2r, be2r, W3p):
    def body(p_ref, u_ref, d_ref, b_ref, g_ref, be_ref, w_ref, out_ref):
        dinv = d_ref[...]
        bia = b_ref[...] * (BN_S * g_ref[...]) + be_ref[...]
        full = jnp.concatenate([p_ref[0] + u_ref[0], p_ref[1] + u_ref[1]],
                               axis=1)
        z = jnp.maximum(dinv * full + bia, 0.0)
        out_ref[...] = dinv * jnp.dot(z, w_ref[...],
                                      preferred_element_type=jnp.float32)

    return pl.pallas_call(
        body,
        out_shape=jax.ShapeDtypeStruct((NPAD, CP), jnp.float32),
    )(p2, u2, dinv, b2r, g2r, be2r, W3p)


def _tc3(p3, u3, dinv, b3r, batch2d):
    def body(p_ref, u_ref, d_ref, b_ref, bt_ref, out_ref):
        dinv = d_ref[pl.ds(0, N)]
        h3 = (dinv * (p_ref[0, pl.ds(0, N)] + p_ref[1, pl.ds(0, N)]
                      + u_ref[pl.ds(0, N)]) + b_ref[...])
        oh = (bt_ref[...] == lax.broadcasted_iota(jnp.int32, (N, G), 1)
              ).astype(jnp.float32)
        psum = lax.dot_general(oh, h3, (((0,), (0,)), ((), ())),
                               preferred_element_type=jnp.float32)
        cnt8 = lax.dot_general(oh, jnp.ones((N, 8), jnp.float32),
                               (((0,), (0,)), ((), ())),
                               preferred_element_type=jnp.float32)
        pooled = psum / jnp.maximum(cnt8[:, 0:1], 1.0)
        valid = lax.broadcasted_iota(jnp.int32, (G, CP), 1) < C
        m = jnp.max(jnp.where(valid, pooled, -1e30), axis=1, keepdims=True)
        ex = jnp.where(valid, jnp.exp(pooled - m), 0.0)
        lse = jnp.log(jnp.sum(ex, axis=1, keepdims=True))
        out_ref[...] = pooled - m - lse

    return pl.pallas_call(
        body,
        out_shape=jax.ShapeDtypeStruct((G, CP), jnp.float32),
    )(p3, u3, dinv, b3r, batch2d)


def kernel(x, edge_index, batch, W1, b1, g1, be1, W2, b2, g2, be2, W3, b3):
    xpad = jnp.pad(x, ((0, NPAD - N), (0, 0)))
    dummy = N + jnp.arange(EP - E, dtype=jnp.int32) % (NPAD - N)
    edge3d = jnp.concatenate(
        [edge_index, jnp.broadcast_to(dummy, (2, EP - E))],
        axis=1).reshape(2, EROWS, K)
    batch2d = batch.reshape(N, 1)
    W3pad = jnp.pad(W3, ((0, 0), (0, CP - C)))
    b3r = jnp.pad(b3, (0, CP - C)).reshape(1, CP)
    b1r, g1r, be1r = b1.reshape(1, H1), g1.reshape(1, H1), be1.reshape(1, H1)
    b2r, g2r, be2r = b2.reshape(1, H2), g2.reshape(1, H2), be2.reshape(1, H2)

    degp = _deg(edge3d)
    u1, dinv = _tc0(xpad, W1, g1r, degp)
    p1 = _PROP1(u1, edge3d)
    u2 = _tc1(p1, u1, dinv, b1r, g1r, be1r, W2, g2r)
    p2 = _PROP2(u2, edge3d)
    u3 = _tc2(p2, u2, dinv, b2r, g2r, be2r, W3pad)
    p3 = _PROP3(u3, edge3d)
    out16 = _tc3(p3, u3, dinv, b3r, batch2d)
    return out16[:, :C]
